# Initial kernel scaffold; baseline (speedup 1.0000x reference)
import jax
import jax.numpy as jnp
from jax.experimental import pallas as pl


def kernel(node_ids, edge_index, edge_type, token_emb, edge_emb_table, W_msg1, b_msg1, W_msg2, b_msg2, W_ih, b_ih, W_hh, b_hh, W_score, b_score):
    raise NotImplementedError("write your pallas kernel here")


# speedup vs baseline: 1.9644x; 1.9644x over previous
import functools

import jax
import jax.numpy as jnp
from jax import lax
from jax.experimental import pallas as pl
from jax.experimental.pallas import tpu as pltpu
from jax.experimental.pallas import tpu_sc as plsc

N_NODES_C = 50000
N_EDGES_C = 800000
EMB_C = 64
MSG_C = 128
EEMB_C = 32
NTOK_C = 32
NET_C = 16
NCOMBO = NTOK_C * NET_C

NW = 32
NODES_PER_TILE = 224
NODES_PER_PASS = NODES_PER_TILE * NW
N_PASSES = 7
N_PAD = NODES_PER_PASS * N_PASSES
HIST_ROWS = NODES_PER_TILE * NCOMBO // 16
KEY_CHUNK = 16000
B_CHUNK = 3200
B_NCHUNK = N_EDGES_C // B_CHUNK


def _mlp_table_kernel(x_ref, w1_ref, b1_ref, w2_ref, b2_ref, o_ref):
    h = jnp.maximum(
        jnp.dot(x_ref[...], w1_ref[...], preferred_element_type=jnp.float32)
        + b1_ref[...], 0.0)
    o_ref[...] = (
        jnp.dot(h, w2_ref[...], preferred_element_type=jnp.float32)
        + b2_ref[...])


def _msg_table(x, w1, b1, w2, b2):
    return pl.pallas_call(
        _mlp_table_kernel,
        out_shape=jax.ShapeDtypeStruct((NCOMBO, MSG_C), jnp.float32),
    )(x, w1, b1.reshape(1, -1), w2, b2.reshape(1, -1))


def _key_kernel(src_hbm, dst_hbm, et_hbm, nid_hbm, key_hbm,
                nid_v, src_v, dst_v, et_v, key_v):
    wid = lax.axis_index("s") * 2 + lax.axis_index("c")
    pltpu.sync_copy(nid_hbm, nid_v)

    def chunk_body(c, carry):
        base = c * B_CHUNK
        pltpu.sync_copy(src_hbm.at[pl.ds(base, B_CHUNK)], src_v)
        pltpu.sync_copy(dst_hbm.at[pl.ds(base, B_CHUNK)], dst_v)
        pltpu.sync_copy(et_hbm.at[pl.ds(base, B_CHUNK)], et_v)

        def vec_body(j, carry2):
            s = src_v[pl.ds(j * 16, 16)]
            tok = plsc.load_gather(nid_v, [s])
            key_v[pl.ds(j * 16, 16)] = (
                dst_v[pl.ds(j * 16, 16)] * 512 + tok * 16
                + et_v[pl.ds(j * 16, 16)])
            return carry2

        lax.fori_loop(0, B_CHUNK // 16, vec_body, 0)
        pltpu.sync_copy(key_v, key_hbm.at[pl.ds(base, B_CHUNK)])
        return carry

    lax.fori_loop(0, (B_NCHUNK - wid + NW - 1) // NW,
                  lambda i, c: chunk_body(wid + i * NW, c), 0)


def _make_key_kernel():
    mesh = plsc.VectorSubcoreMesh(core_axis_name="c", subcore_axis_name="s")
    return functools.partial(
        pl.kernel,
        mesh=mesh,
        out_type=jax.ShapeDtypeStruct((N_EDGES_C,), jnp.int32),
        scratch_types=[
            pltpu.VMEM((N_NODES_C,), jnp.int32),
            pltpu.VMEM((B_CHUNK,), jnp.int32),
            pltpu.VMEM((B_CHUNK,), jnp.int32),
            pltpu.VMEM((B_CHUNK,), jnp.int32),
            pltpu.VMEM((B_CHUNK,), jnp.int32),
        ],
        compiler_params=pltpu.CompilerParams(needs_layout_passes=False, use_tc_tiling_on_sc=False),
    )(_key_kernel)


def _hist_kernel(key_hbm, h_hbm, hist_v, key_v):
    wid = lax.axis_index("s") * 2 + lax.axis_index("c")
    ones = jnp.full((16,), 1.0, jnp.float32)
    zeros = jnp.zeros((16,), jnp.float32)

    def one_pass(p, carry):
        lo512 = (p * NW + wid) * NODES_PER_TILE * 512

        def zero_body(j, c2):
            hist_v[j] = zeros
            return c2

        lax.fori_loop(0, HIST_ROWS, zero_body, 0)

        def chunk_body(c, c2):
            pltpu.sync_copy(key_hbm.at[pl.ds(c * KEY_CHUNK, KEY_CHUNK)],
                            key_v)

            def vec_body(j, c3):
                rel = key_v[pl.ds(j * 16, 16)] - lo512
                msk = (rel >= 0) & (rel < NODES_PER_TILE * 512)
                row = lax.shift_right_logical(rel, 4)
                col = rel & 15
                plsc.addupdate_scatter(hist_v, [row, col], ones, mask=msk)
                return c3

            lax.fori_loop(0, KEY_CHUNK // 16, vec_body, 0)
            return c2

        lax.fori_loop(0, N_EDGES_C // KEY_CHUNK, chunk_body, 0)
        out_row = (p * NW + wid) * HIST_ROWS
        pltpu.sync_copy(hist_v, h_hbm.at[pl.ds(out_row, HIST_ROWS)])
        return carry

    lax.fori_loop(0, N_PASSES, one_pass, 0)


def _make_hist_kernel():
    mesh = plsc.VectorSubcoreMesh(core_axis_name="c", subcore_axis_name="s")
    return functools.partial(
        pl.kernel,
        mesh=mesh,
        out_type=jax.ShapeDtypeStruct((N_PAD * 32, 16), jnp.float32),
        scratch_types=[
            pltpu.VMEM((HIST_ROWS, 16), jnp.float32),
            pltpu.VMEM((KEY_CHUNK,), jnp.int32),
        ],
        compiler_params=pltpu.CompilerParams(needs_layout_passes=False, use_tc_tiling_on_sc=False),
    )(_hist_kernel)


def _merge_kernel(h_ref, nid_ref, m_ref, temb_ref,
                  wir_ref, wiz_ref, win_ref, bir_ref, biz_ref, bin_ref,
                  whr_ref, whz_ref, whn_ref, bhr_ref, bhz_ref, bhn_ref,
                  ws_ref, bs_ref, o_ref):
    hmat = h_ref[...]
    cnt = jnp.sum(hmat, axis=1, keepdims=True)
    msum = jnp.dot(hmat, m_ref[...], preferred_element_type=jnp.float32)
    magg = msum / jnp.maximum(cnt, 1.0)

    nid = nid_ref[0]
    onehot = (nid == lax.broadcasted_iota(
        jnp.int32, (nid.shape[0], NTOK_C), 1)).astype(jnp.float32)
    h = jnp.dot(onehot, temb_ref[...], preferred_element_type=jnp.float32)

    def gi(w_ref, b_ref):
        return jnp.dot(magg, w_ref[...],
                       preferred_element_type=jnp.float32) + b_ref[...]

    def gh(w_ref, b_ref):
        return jnp.dot(h, w_ref[...],
                       preferred_element_type=jnp.float32) + b_ref[...]

    r = jax.nn.sigmoid(gi(wir_ref, bir_ref) + gh(whr_ref, bhr_ref))
    z = jax.nn.sigmoid(gi(wiz_ref, biz_ref) + gh(whz_ref, bhz_ref))
    n = jnp.tanh(gi(win_ref, bin_ref) + r * gh(whn_ref, bhn_ref))
    h_new = (1.0 - z) * n + z * h

    logit = jnp.sum(h_new * ws_ref[...], axis=1, keepdims=True) + bs_ref[...]
    o_ref[0] = logit


def _merge(h_mat, node_ids3, m_tab, token_emb, w_ih, b_ih, w_hh, b_hh,
           w_score, b_score):
    bn = 2000
    nblk = N_NODES_C // bn
    full = lambda i: (0, 0)
    wi = [w_ih[:, k * EMB_C:(k + 1) * EMB_C] for k in range(3)]
    bi = [b_ih[k * EMB_C:(k + 1) * EMB_C].reshape(1, -1) for k in range(3)]
    wh = [w_hh[:, k * EMB_C:(k + 1) * EMB_C] for k in range(3)]
    bh = [b_hh[k * EMB_C:(k + 1) * EMB_C].reshape(1, -1) for k in range(3)]
    out = pl.pallas_call(
        _merge_kernel,
        grid=(nblk,),
        in_specs=[
            pl.BlockSpec((bn, NCOMBO), lambda i: (i, 0)),
            pl.BlockSpec((1, bn, 1), lambda i: (i, 0, 0)),
            pl.BlockSpec((NCOMBO, MSG_C), full),
            pl.BlockSpec((NTOK_C, EMB_C), full),
            pl.BlockSpec((MSG_C, EMB_C), full),
            pl.BlockSpec((MSG_C, EMB_C), full),
            pl.BlockSpec((MSG_C, EMB_C), full),
            pl.BlockSpec((1, EMB_C), full),
            pl.BlockSpec((1, EMB_C), full),
            pl.BlockSpec((1, EMB_C), full),
            pl.BlockSpec((EMB_C, EMB_C), full),
            pl.BlockSpec((EMB_C, EMB_C), full),
            pl.BlockSpec((EMB_C, EMB_C), full),
            pl.BlockSpec((1, EMB_C), full),
            pl.BlockSpec((1, EMB_C), full),
            pl.BlockSpec((1, EMB_C), full),
            pl.BlockSpec((1, EMB_C), full),
            pl.BlockSpec((1, 1), full),
        ],
        out_specs=pl.BlockSpec((1, bn, 1), lambda i: (i, 0, 0)),
        out_shape=jax.ShapeDtypeStruct((nblk, bn, 1), jnp.float32),
    )(h_mat, node_ids3, m_tab, token_emb,
      wi[0], wi[1], wi[2], bi[0], bi[1], bi[2],
      wh[0], wh[1], wh[2], bh[0], bh[1], bh[2],
      w_score.reshape(1, EMB_C), b_score.reshape(1, 1))
    return out.reshape(N_NODES_C)


@jax.jit
def kernel(node_ids, edge_index, edge_type, token_emb, edge_emb_table,
           W_msg1, b_msg1, W_msg2, b_msg2, W_ih, b_ih, W_hh, b_hh,
           W_score, b_score):
    node_ids = node_ids.astype(jnp.int32)
    src = edge_index[0].astype(jnp.int32)
    dst = edge_index[1].astype(jnp.int32)
    et = edge_type.astype(jnp.int32)

    x = jnp.concatenate(
        [jnp.repeat(token_emb, NET_C, axis=0),
         jnp.tile(edge_emb_table, (NTOK_C, 1))], axis=1)
    m_tab = _msg_table(x, W_msg1, b_msg1, W_msg2, b_msg2)

    key = _make_key_kernel()(src, dst, et, node_ids)

    h_raw = _make_hist_kernel()(key)
    h_mat = h_raw.reshape(N_PAD, NCOMBO)

    nid3 = node_ids.reshape(N_NODES_C // 2000, 2000, 1)
    return _merge(h_mat, nid3, m_tab, token_emb, W_ih, b_ih, W_hh, b_hh,
                  W_score, b_score)


# speedup vs baseline: 2.1371x; 1.0879x over previous
import functools

import jax
import jax.numpy as jnp
from jax import lax
from jax.experimental import pallas as pl
from jax.experimental.pallas import tpu as pltpu
from jax.experimental.pallas import tpu_sc as plsc

N_NODES_C = 50000
N_EDGES_C = 800000
EMB_C = 64
MSG_C = 128
EEMB_C = 32
NTOK_C = 32
NET_C = 16
NCOMBO = NTOK_C * NET_C

NW = 32
NODES_PER_TILE = 224
NODES_PER_PASS = NODES_PER_TILE * NW
N_PASSES = 7
N_PAD = NODES_PER_PASS * N_PASSES
HIST_ROWS = NODES_PER_TILE * NCOMBO // 16
KEY_CHUNK = 16000
B_CHUNK = 3200
B_NCHUNK = N_EDGES_C // B_CHUNK


def _mlp_table_kernel(x_ref, w1_ref, b1_ref, w2_ref, b2_ref, o_ref):
    h = jnp.maximum(
        jnp.dot(x_ref[...], w1_ref[...], preferred_element_type=jnp.float32)
        + b1_ref[...], 0.0)
    o_ref[...] = (
        jnp.dot(h, w2_ref[...], preferred_element_type=jnp.float32)
        + b2_ref[...])


def _msg_table(x, w1, b1, w2, b2):
    return pl.pallas_call(
        _mlp_table_kernel,
        out_shape=jax.ShapeDtypeStruct((NCOMBO, MSG_C), jnp.float32),
    )(x, w1, b1.reshape(1, -1), w2, b2.reshape(1, -1))


def _key_kernel(src_hbm, dst_hbm, et_hbm, nid_hbm, key_hbm,
                nid_v, src_v, dst_v, et_v, key_v):
    wid = lax.axis_index("s") * 2 + lax.axis_index("c")
    pltpu.sync_copy(nid_hbm, nid_v)

    def chunk_body(c, carry):
        base = c * B_CHUNK
        pltpu.sync_copy(src_hbm.at[pl.ds(base, B_CHUNK)], src_v)
        pltpu.sync_copy(dst_hbm.at[pl.ds(base, B_CHUNK)], dst_v)
        pltpu.sync_copy(et_hbm.at[pl.ds(base, B_CHUNK)], et_v)

        def vec_body(j, carry2):
            s = src_v[pl.ds(j * 16, 16)]
            tok = plsc.load_gather(nid_v, [s])
            key_v[pl.ds(j * 16, 16)] = (
                dst_v[pl.ds(j * 16, 16)] * 512 + tok * 16
                + et_v[pl.ds(j * 16, 16)])
            return carry2

        lax.fori_loop(0, B_CHUNK // 16, vec_body, 0)
        pltpu.sync_copy(key_v, key_hbm.at[pl.ds(base, B_CHUNK)])
        return carry

    lax.fori_loop(0, (B_NCHUNK - wid + NW - 1) // NW,
                  lambda i, c: chunk_body(wid + i * NW, c), 0)


def _make_key_kernel():
    mesh = plsc.VectorSubcoreMesh(core_axis_name="c", subcore_axis_name="s")
    return functools.partial(
        pl.kernel,
        mesh=mesh,
        out_type=jax.ShapeDtypeStruct((N_EDGES_C,), jnp.int32),
        scratch_types=[
            pltpu.VMEM((N_NODES_C,), jnp.int32),
            pltpu.VMEM((B_CHUNK,), jnp.int32),
            pltpu.VMEM((B_CHUNK,), jnp.int32),
            pltpu.VMEM((B_CHUNK,), jnp.int32),
            pltpu.VMEM((B_CHUNK,), jnp.int32),
        ],
        compiler_params=pltpu.CompilerParams(needs_layout_passes=False, use_tc_tiling_on_sc=False),
    )(_key_kernel)


def _hist_kernel(key_hbm, h_hbm, hist_v, key_v):
    wid = lax.axis_index("s") * 2 + lax.axis_index("c")
    ones = jnp.full((16,), 1.0, jnp.float32)
    zeros = jnp.zeros((16,), jnp.float32)

    def one_pass(p, carry):
        lo512 = (p * NW + wid) * NODES_PER_TILE * 512

        def zero_body(j, c2):
            hist_v[j] = zeros
            return c2

        lax.fori_loop(0, HIST_ROWS, zero_body, 0)

        def chunk_body(c, c2):
            pltpu.sync_copy(key_hbm.at[pl.ds(c * KEY_CHUNK, KEY_CHUNK)],
                            key_v)

            def vec_body(j, c3):
                for u in range(8):
                    rel = key_v[pl.ds(j * 128 + u * 16, 16)] - lo512
                    msk = (rel >= 0) & (rel < NODES_PER_TILE * 512)
                    row = lax.shift_right_logical(rel, 4)
                    col = rel & 15
                    plsc.addupdate_scatter(hist_v, [row, col], ones,
                                           mask=msk)
                return c3

            lax.fori_loop(0, KEY_CHUNK // 128, vec_body, 0)
            return c2

        lax.fori_loop(0, N_EDGES_C // KEY_CHUNK, chunk_body, 0)
        out_row = (p * NW + wid) * HIST_ROWS
        pltpu.sync_copy(hist_v, h_hbm.at[pl.ds(out_row, HIST_ROWS)])
        return carry

    lax.fori_loop(0, N_PASSES, one_pass, 0)


def _make_hist_kernel():
    mesh = plsc.VectorSubcoreMesh(core_axis_name="c", subcore_axis_name="s")
    return functools.partial(
        pl.kernel,
        mesh=mesh,
        out_type=jax.ShapeDtypeStruct((N_PAD * 32, 16), jnp.float32),
        scratch_types=[
            pltpu.VMEM((HIST_ROWS, 16), jnp.float32),
            pltpu.VMEM((KEY_CHUNK,), jnp.int32),
        ],
        compiler_params=pltpu.CompilerParams(needs_layout_passes=False, use_tc_tiling_on_sc=False),
    )(_hist_kernel)


def _merge_kernel(h_ref, nid_ref, m_ref, temb_ref,
                  wir_ref, wiz_ref, win_ref, bir_ref, biz_ref, bin_ref,
                  whr_ref, whz_ref, whn_ref, bhr_ref, bhz_ref, bhn_ref,
                  ws_ref, bs_ref, o_ref):
    hmat = h_ref[...]
    cnt = jnp.sum(hmat, axis=1, keepdims=True)
    msum = jnp.dot(hmat, m_ref[...], preferred_element_type=jnp.float32)
    magg = msum / jnp.maximum(cnt, 1.0)

    nid = nid_ref[0]
    onehot = (nid == lax.broadcasted_iota(
        jnp.int32, (nid.shape[0], NTOK_C), 1)).astype(jnp.float32)
    h = jnp.dot(onehot, temb_ref[...], preferred_element_type=jnp.float32)

    def gi(w_ref, b_ref):
        return jnp.dot(magg, w_ref[...],
                       preferred_element_type=jnp.float32) + b_ref[...]

    def gh(w_ref, b_ref):
        return jnp.dot(h, w_ref[...],
                       preferred_element_type=jnp.float32) + b_ref[...]

    r = jax.nn.sigmoid(gi(wir_ref, bir_ref) + gh(whr_ref, bhr_ref))
    z = jax.nn.sigmoid(gi(wiz_ref, biz_ref) + gh(whz_ref, bhz_ref))
    n = jnp.tanh(gi(win_ref, bin_ref) + r * gh(whn_ref, bhn_ref))
    h_new = (1.0 - z) * n + z * h

    logit = jnp.sum(h_new * ws_ref[...], axis=1, keepdims=True) + bs_ref[...]
    o_ref[0] = logit


def _merge(h_mat, node_ids3, m_tab, token_emb, w_ih, b_ih, w_hh, b_hh,
           w_score, b_score):
    bn = 2000
    nblk = N_NODES_C // bn
    full = lambda i: (0, 0)
    wi = [w_ih[:, k * EMB_C:(k + 1) * EMB_C] for k in range(3)]
    bi = [b_ih[k * EMB_C:(k + 1) * EMB_C].reshape(1, -1) for k in range(3)]
    wh = [w_hh[:, k * EMB_C:(k + 1) * EMB_C] for k in range(3)]
    bh = [b_hh[k * EMB_C:(k + 1) * EMB_C].reshape(1, -1) for k in range(3)]
    out = pl.pallas_call(
        _merge_kernel,
        grid=(nblk,),
        in_specs=[
            pl.BlockSpec((bn, NCOMBO), lambda i: (i, 0)),
            pl.BlockSpec((1, bn, 1), lambda i: (i, 0, 0)),
            pl.BlockSpec((NCOMBO, MSG_C), full),
            pl.BlockSpec((NTOK_C, EMB_C), full),
            pl.BlockSpec((MSG_C, EMB_C), full),
            pl.BlockSpec((MSG_C, EMB_C), full),
            pl.BlockSpec((MSG_C, EMB_C), full),
            pl.BlockSpec((1, EMB_C), full),
            pl.BlockSpec((1, EMB_C), full),
            pl.BlockSpec((1, EMB_C), full),
            pl.BlockSpec((EMB_C, EMB_C), full),
            pl.BlockSpec((EMB_C, EMB_C), full),
            pl.BlockSpec((EMB_C, EMB_C), full),
            pl.BlockSpec((1, EMB_C), full),
            pl.BlockSpec((1, EMB_C), full),
            pl.BlockSpec((1, EMB_C), full),
            pl.BlockSpec((1, EMB_C), full),
            pl.BlockSpec((1, 1), full),
        ],
        out_specs=pl.BlockSpec((1, bn, 1), lambda i: (i, 0, 0)),
        out_shape=jax.ShapeDtypeStruct((nblk, bn, 1), jnp.float32),
    )(h_mat, node_ids3, m_tab, token_emb,
      wi[0], wi[1], wi[2], bi[0], bi[1], bi[2],
      wh[0], wh[1], wh[2], bh[0], bh[1], bh[2],
      w_score.reshape(1, EMB_C), b_score.reshape(1, 1))
    return out.reshape(N_NODES_C)


@jax.jit
def kernel(node_ids, edge_index, edge_type, token_emb, edge_emb_table,
           W_msg1, b_msg1, W_msg2, b_msg2, W_ih, b_ih, W_hh, b_hh,
           W_score, b_score):
    node_ids = node_ids.astype(jnp.int32)
    src = edge_index[0].astype(jnp.int32)
    dst = edge_index[1].astype(jnp.int32)
    et = edge_type.astype(jnp.int32)

    x = jnp.concatenate(
        [jnp.repeat(token_emb, NET_C, axis=0),
         jnp.tile(edge_emb_table, (NTOK_C, 1))], axis=1)
    m_tab = _msg_table(x, W_msg1, b_msg1, W_msg2, b_msg2)

    key = _make_key_kernel()(src, dst, et, node_ids)

    h_raw = _make_hist_kernel()(key)
    h_mat = h_raw.reshape(N_PAD, NCOMBO)

    nid3 = node_ids.reshape(N_NODES_C // 2000, 2000, 1)
    return _merge(h_mat, nid3, m_tab, token_emb, W_ih, b_ih, W_hh, b_hh,
                  W_score, b_score)


# speedup vs baseline: 5.5618x; 2.6025x over previous
import functools

import jax
import jax.numpy as jnp
from jax import lax
from jax.experimental import pallas as pl
from jax.experimental.pallas import tpu as pltpu
from jax.experimental.pallas import tpu_sc as plsc

N_NODES_C = 50000
N_EDGES_C = 800000
EMB_C = 64
MSG_C = 128
EEMB_C = 32
NTOK_C = 32
NET_C = 16
NCOMBO = NTOK_C * NET_C

NW = 32
NODES_PER_TILE = 224
NODES_PER_PASS = NODES_PER_TILE * NW
N_PASSES = 7
N_PAD = NODES_PER_PASS * N_PASSES
HIST_ROWS = NODES_PER_TILE * NCOMBO // 16
KEY_CHUNK = 16000
B_CHUNK = 3200
B_NCHUNK = N_EDGES_C // B_CHUNK


def _mlp_table_kernel(x_ref, w1_ref, b1_ref, w2_ref, b2_ref, o_ref):
    h = jnp.maximum(
        jnp.dot(x_ref[...], w1_ref[...], preferred_element_type=jnp.float32)
        + b1_ref[...], 0.0)
    o_ref[...] = (
        jnp.dot(h, w2_ref[...], preferred_element_type=jnp.float32)
        + b2_ref[...])


def _msg_table(x, w1, b1, w2, b2):
    return pl.pallas_call(
        _mlp_table_kernel,
        out_shape=jax.ShapeDtypeStruct((NCOMBO, MSG_C), jnp.float32),
    )(x, w1, b1.reshape(1, -1), w2, b2.reshape(1, -1))


def _key_kernel(src_hbm, dst_hbm, et_hbm, nid_hbm, key_hbm,
                nid_v, src_v, dst_v, et_v, key_v):
    wid = lax.axis_index("s") * 2 + lax.axis_index("c")
    pltpu.sync_copy(nid_hbm, nid_v)

    def chunk_body(c, carry):
        base = c * B_CHUNK
        pltpu.sync_copy(src_hbm.at[pl.ds(base, B_CHUNK)], src_v)
        pltpu.sync_copy(dst_hbm.at[pl.ds(base, B_CHUNK)], dst_v)
        pltpu.sync_copy(et_hbm.at[pl.ds(base, B_CHUNK)], et_v)

        def vec_body(j, carry2):
            s = src_v[pl.ds(j * 16, 16)]
            tok = plsc.load_gather(nid_v, [s])
            key_v[pl.ds(j * 16, 16)] = (
                dst_v[pl.ds(j * 16, 16)] * 512 + tok * 16
                + et_v[pl.ds(j * 16, 16)])
            return carry2

        lax.fori_loop(0, B_CHUNK // 16, vec_body, 0)
        pltpu.sync_copy(key_v, key_hbm.at[pl.ds(base, B_CHUNK)])
        return carry

    lax.fori_loop(0, (B_NCHUNK - wid + NW - 1) // NW,
                  lambda i, c: chunk_body(wid + i * NW, c), 0)


def _make_key_kernel():
    mesh = plsc.VectorSubcoreMesh(core_axis_name="c", subcore_axis_name="s")
    return functools.partial(
        pl.kernel,
        mesh=mesh,
        out_type=jax.ShapeDtypeStruct((N_EDGES_C,), jnp.int32),
        scratch_types=[
            pltpu.VMEM((N_NODES_C,), jnp.int32),
            pltpu.VMEM((B_CHUNK,), jnp.int32),
            pltpu.VMEM((B_CHUNK,), jnp.int32),
            pltpu.VMEM((B_CHUNK,), jnp.int32),
            pltpu.VMEM((B_CHUNK,), jnp.int32),
        ],
        compiler_params=pltpu.CompilerParams(needs_layout_passes=False, use_tc_tiling_on_sc=False),
    )(_key_kernel)


def _hist_kernel(key_hbm, h_hbm, hist_v, key_v):
    wid = lax.axis_index("s") * 2 + lax.axis_index("c")
    ones = jnp.full((16,), 1.0, jnp.float32)
    zeros = jnp.zeros((16,), jnp.float32)

    def one_pass(p, carry):
        lo512 = (p * NW + wid) * NODES_PER_TILE * 512

        def zero_body(j, c2):
            hist_v[j] = zeros
            return c2

        lax.fori_loop(0, HIST_ROWS, zero_body, 0)

        def chunk_body(c, c2):
            pltpu.sync_copy(key_hbm.at[pl.ds(c * KEY_CHUNK, KEY_CHUNK)],
                            key_v)

            @plsc.parallel_loop(0, KEY_CHUNK // 16, unroll=8)
            def vec_body(j):
                rel = key_v[pl.ds(j * 16, 16)] - lo512
                msk = (rel >= 0) & (rel < NODES_PER_TILE * 512)
                row = lax.shift_right_logical(rel, 4)
                col = rel & 15
                plsc.addupdate_scatter(hist_v, [row, col], ones, mask=msk)

            return c2

        lax.fori_loop(0, N_EDGES_C // KEY_CHUNK, chunk_body, 0)
        out_row = (p * NW + wid) * HIST_ROWS
        pltpu.sync_copy(hist_v, h_hbm.at[pl.ds(out_row, HIST_ROWS)])
        return carry

    lax.fori_loop(0, N_PASSES, one_pass, 0)


def _make_hist_kernel():
    mesh = plsc.VectorSubcoreMesh(core_axis_name="c", subcore_axis_name="s")
    return functools.partial(
        pl.kernel,
        mesh=mesh,
        out_type=jax.ShapeDtypeStruct((N_PAD * 32, 16), jnp.float32),
        scratch_types=[
            pltpu.VMEM((HIST_ROWS, 16), jnp.float32),
            pltpu.VMEM((KEY_CHUNK,), jnp.int32),
        ],
        compiler_params=pltpu.CompilerParams(needs_layout_passes=False, use_tc_tiling_on_sc=False),
    )(_hist_kernel)


def _merge_kernel(h_ref, nid_ref, m_ref, temb_ref,
                  wir_ref, wiz_ref, win_ref, bir_ref, biz_ref, bin_ref,
                  whr_ref, whz_ref, whn_ref, bhr_ref, bhz_ref, bhn_ref,
                  ws_ref, bs_ref, o_ref):
    hmat = h_ref[...]
    cnt = jnp.sum(hmat, axis=1, keepdims=True)
    msum = jnp.dot(hmat, m_ref[...], preferred_element_type=jnp.float32)
    magg = msum / jnp.maximum(cnt, 1.0)

    nid = nid_ref[0]
    onehot = (nid == lax.broadcasted_iota(
        jnp.int32, (nid.shape[0], NTOK_C), 1)).astype(jnp.float32)
    h = jnp.dot(onehot, temb_ref[...], preferred_element_type=jnp.float32)

    def gi(w_ref, b_ref):
        return jnp.dot(magg, w_ref[...],
                       preferred_element_type=jnp.float32) + b_ref[...]

    def gh(w_ref, b_ref):
        return jnp.dot(h, w_ref[...],
                       preferred_element_type=jnp.float32) + b_ref[...]

    r = jax.nn.sigmoid(gi(wir_ref, bir_ref) + gh(whr_ref, bhr_ref))
    z = jax.nn.sigmoid(gi(wiz_ref, biz_ref) + gh(whz_ref, bhz_ref))
    n = jnp.tanh(gi(win_ref, bin_ref) + r * gh(whn_ref, bhn_ref))
    h_new = (1.0 - z) * n + z * h

    logit = jnp.sum(h_new * ws_ref[...], axis=1, keepdims=True) + bs_ref[...]
    o_ref[0] = logit


def _merge(h_mat, node_ids3, m_tab, token_emb, w_ih, b_ih, w_hh, b_hh,
           w_score, b_score):
    bn = 2000
    nblk = N_NODES_C // bn
    full = lambda i: (0, 0)
    wi = [w_ih[:, k * EMB_C:(k + 1) * EMB_C] for k in range(3)]
    bi = [b_ih[k * EMB_C:(k + 1) * EMB_C].reshape(1, -1) for k in range(3)]
    wh = [w_hh[:, k * EMB_C:(k + 1) * EMB_C] for k in range(3)]
    bh = [b_hh[k * EMB_C:(k + 1) * EMB_C].reshape(1, -1) for k in range(3)]
    out = pl.pallas_call(
        _merge_kernel,
        grid=(nblk,),
        in_specs=[
            pl.BlockSpec((bn, NCOMBO), lambda i: (i, 0)),
            pl.BlockSpec((1, bn, 1), lambda i: (i, 0, 0)),
            pl.BlockSpec((NCOMBO, MSG_C), full),
            pl.BlockSpec((NTOK_C, EMB_C), full),
            pl.BlockSpec((MSG_C, EMB_C), full),
            pl.BlockSpec((MSG_C, EMB_C), full),
            pl.BlockSpec((MSG_C, EMB_C), full),
            pl.BlockSpec((1, EMB_C), full),
            pl.BlockSpec((1, EMB_C), full),
            pl.BlockSpec((1, EMB_C), full),
            pl.BlockSpec((EMB_C, EMB_C), full),
            pl.BlockSpec((EMB_C, EMB_C), full),
            pl.BlockSpec((EMB_C, EMB_C), full),
            pl.BlockSpec((1, EMB_C), full),
            pl.BlockSpec((1, EMB_C), full),
            pl.BlockSpec((1, EMB_C), full),
            pl.BlockSpec((1, EMB_C), full),
            pl.BlockSpec((1, 1), full),
        ],
        out_specs=pl.BlockSpec((1, bn, 1), lambda i: (i, 0, 0)),
        out_shape=jax.ShapeDtypeStruct((nblk, bn, 1), jnp.float32),
    )(h_mat, node_ids3, m_tab, token_emb,
      wi[0], wi[1], wi[2], bi[0], bi[1], bi[2],
      wh[0], wh[1], wh[2], bh[0], bh[1], bh[2],
      w_score.reshape(1, EMB_C), b_score.reshape(1, 1))
    return out.reshape(N_NODES_C)


@jax.jit
def kernel(node_ids, edge_index, edge_type, token_emb, edge_emb_table,
           W_msg1, b_msg1, W_msg2, b_msg2, W_ih, b_ih, W_hh, b_hh,
           W_score, b_score):
    node_ids = node_ids.astype(jnp.int32)
    src = edge_index[0].astype(jnp.int32)
    dst = edge_index[1].astype(jnp.int32)
    et = edge_type.astype(jnp.int32)

    x = jnp.concatenate(
        [jnp.repeat(token_emb, NET_C, axis=0),
         jnp.tile(edge_emb_table, (NTOK_C, 1))], axis=1)
    m_tab = _msg_table(x, W_msg1, b_msg1, W_msg2, b_msg2)

    key = _make_key_kernel()(src, dst, et, node_ids)

    h_raw = _make_hist_kernel()(key)
    h_mat = h_raw.reshape(N_PAD, NCOMBO)

    nid3 = node_ids.reshape(N_NODES_C // 2000, 2000, 1)
    return _merge(h_mat, nid3, m_tab, token_emb, W_ih, b_ih, W_hh, b_hh,
                  W_score, b_score)
